# baseline (speedup 1.0000x reference)
import functools

import jax
import jax.numpy as jnp
from jax import lax
from jax.experimental import pallas as pl
from jax.experimental.pallas import tpu as pltpu
from jax.experimental.pallas import tpu_sc as plsc

N = 10000; E = 320000; NG = 1000; NM = 500; E2 = 100000
D0 = 240; D1 = 256; H = 4; NB = 64; CUT = 5.0; LCUT = 9.0
AVG_DEG = 15.57930850982666; AVG_NODES = 18.03065905448718

NC = 2
NS = 16
NW = NC * NS
CH = 128


def _silu(x):
    return x * jax.nn.sigmoid(x)


def _ln(x):
    m = x.mean(-1, keepdims=True)
    v = x.var(-1, keepdims=True)
    return (x - m) / jnp.sqrt(v + 1e-5)


def _sh(u):
    x, y, z = u[:, 0], u[:, 1], u[:, 2]
    s3 = jnp.sqrt(3.0); s15 = jnp.sqrt(15.0); s5 = jnp.sqrt(5.0)
    return jnp.stack([jnp.ones_like(x), s3 * x, s3 * y, s3 * z,
                      s15 * x * y, s15 * y * z, 0.5 * s5 * (3.0 * z * z - 1.0),
                      s15 * x * z, 0.5 * s15 * (x * x - y * y)], axis=-1)


def _rbf(d):
    c = jnp.linspace(0.0, CUT, NB)
    w = CUT / NB
    return jnp.exp(-(((d[:, None] - c[None, :]) / w) ** 2))


def _round_up(a, b):
    return (a + b - 1) // b * b


def _seg_sum_body(pw, n_pad, d, vals, idx, zeros, out,
                  iv0, iv1, rv0, rv1, si0, si1, sr0, sr1, acc):
    c = lax.axis_index("c")
    s = lax.axis_index("s")
    wid = s * NC + c
    base_row = wid * (pw * CH)

    rpw = n_pad // NS
    r0 = s * rpw
    pltpu.sync_copy(zeros.at[pl.ds(r0, rpw)], acc.at[pl.ds(r0, rpw)])
    plsc.subcore_barrier()

    ivs = (iv0, iv1); rvs = (rv0, rv1); sis = (si0, si1); srs = (sr0, sr1)

    def start(g, b):
        row = base_row + g * CH
        pltpu.async_copy(idx.at[pl.ds(row, CH)], ivs[b], sis[b])
        pltpu.async_copy(vals.at[pl.ds(row, CH)], rvs[b], srs[b])

    def wait(g, b):
        row = base_row + g * CH
        pltpu.make_async_copy(idx.at[pl.ds(row, CH)], ivs[b], sis[b]).wait()
        pltpu.make_async_copy(vals.at[pl.ds(row, CH)], rvs[b], srs[b]).wait()

    def scat(b):
        pltpu.sync_copy(rvs[b], acc.at[ivs[b]], add=True)

    nit = pw // 2
    start(0, 0)

    def body(t, carry):
        g0 = t * 2
        wait(g0, 0)
        start(g0 + 1, 1)
        scat(0)
        wait(g0 + 1, 1)

        @pl.when(t < nit - 1)
        def _():
            start(g0 + 2, 0)

        scat(1)
        return carry

    lax.fori_loop(0, nit, body, 0)

    plsc.subcore_barrier()
    pltpu.sync_copy(acc.at[pl.ds(r0, rpw)], out.at[pl.ds(c * n_pad + r0, rpw)])


def _sc_segment_sum_1(values, idx, n_out):
    ep, d = values.shape
    assert d in (16, 128) and ep % (NW * CH * 2) == 0
    n_pad = _round_up(n_out, NS * 8)
    pw = ep // (NW * CH)
    zeros = jnp.zeros((n_pad, d), jnp.float32)

    mesh = plsc.VectorSubcoreMesh(core_axis_name="c", subcore_axis_name="s")
    k = pl.kernel(
        functools.partial(_seg_sum_body, pw, n_pad, d),
        mesh=mesh,
        out_type=jax.ShapeDtypeStruct((NC * n_pad, d), jnp.float32),
        scratch_types=[
            pltpu.VMEM((CH,), jnp.int32),
            pltpu.VMEM((CH,), jnp.int32),
            pltpu.VMEM((CH, d), jnp.float32),
            pltpu.VMEM((CH, d), jnp.float32),
            pltpu.SemaphoreType.DMA,
            pltpu.SemaphoreType.DMA,
            pltpu.SemaphoreType.DMA,
            pltpu.SemaphoreType.DMA,
            pltpu.VMEM_SHARED((n_pad, d), jnp.float32),
        ],
    )
    out = k(values, idx.astype(jnp.int32), zeros)
    out = out.reshape(NC, n_pad, d)
    return (out[0] + out[1])[:n_out]


def _sc_segment_sum(values, idx, n_out):
    e0, d = values.shape
    dp = 16 if d <= 16 else _round_up(d, 128)
    ep = _round_up(e0, NW * CH * 2)
    if dp != d or ep != e0:
        values = jnp.pad(values, ((0, ep - e0), (0, dp - d)))
    if ep != e0:
        idx = jnp.pad(idx, (0, ep - e0))
    idx = idx.astype(jnp.int32)
    parts = []
    for c0 in range(0, dp, 128):
        w = min(128, dp - c0)
        parts.append(_sc_segment_sum_1(values[:, c0:c0 + w], idx, n_out))
    return jnp.concatenate(parts, axis=1)[:, :d]


def _attn(xin, rbf, src, dst, p, n_out):
    q = xin @ p['Wq']; k = xin @ p['Wk']; v = xin @ p['Wv']
    Ds = q.shape[1] // H; Dv = v.shape[1] // H
    gate = _silu(rbf @ p['Wg1']) @ p['Wg2']
    qe = q[dst].reshape(-1, H, Ds); ke = k[src].reshape(-1, H, Ds)
    logit = jnp.sum(qe * ke, -1) / jnp.sqrt(float(Ds)) + gate
    qn = jnp.sqrt(jnp.sum(q.reshape(n_out, H, Ds) ** 2, -1))
    kmax = jnp.sqrt(jnp.max(jnp.sum(k.reshape(-1, H, Ds) ** 2, -1), axis=0))
    mhat = qn * kmax[None, :] / jnp.sqrt(float(Ds)) + jnp.max(gate, axis=0)[None, :]
    a = jnp.exp(logit - mhat[dst])
    u = (v[src].reshape(-1, H, Dv) * a[..., None]).reshape(-1, H * Dv)
    su = _sc_segment_sum(jnp.concatenate([a, u], axis=1), dst, n_out)
    den = su[:, :H] + 1e-9
    num = su[:, H:]
    return (num.reshape(n_out, H, Dv) / den[..., None]).reshape(n_out, H * Dv) @ p['Wo']


def _readout_body(h_ref, w1_ref, w2_ref, o_ref):
    t = h_ref[...] @ w1_ref[...]
    t = t * jax.nn.sigmoid(t)
    o_ref[...] = t @ w2_ref[...]


def _readout(h, w1, w2):
    BLK = 400
    return pl.pallas_call(
        _readout_body,
        grid=(N // BLK,),
        in_specs=[
            pl.BlockSpec((BLK, 2 * D1), lambda i: (i, 0)),
            pl.BlockSpec((2 * D1, D1), lambda i: (0, 0)),
            pl.BlockSpec((D1, 1), lambda i: (0, 0)),
        ],
        out_specs=pl.BlockSpec((BLK, 1), lambda i: (i, 0)),
        out_shape=jax.ShapeDtypeStruct((N, 1), jnp.float32),
    )(h, w1, w2)


def kernel(pos, params, batch, labels, atomic_numbers, edge_index, interaction_graph):
    z = atomic_numbers.astype(jnp.float32)[:, None]
    gpos = jax.ops.segment_sum(pos * z, labels, num_segments=NG) / (jax.ops.segment_sum(z, labels, num_segments=NG) + 1e-6)
    nid, gid = interaction_graph[0], interaction_graph[1]
    ngd = jnp.sqrt(jnp.sum((pos[nid] - gpos[gid]) ** 2, -1) + 1e-12)
    lmask = (ngd <= LCUT)
    src, dst = edge_index[0], edge_index[1]
    evec = pos[src] - pos[dst]
    elen = jnp.sqrt(jnp.sum(evec ** 2, -1) + 1e-12)
    sh = _sh(evec / elen[:, None])
    rbf = _rbf(elen)
    x = params['W_atom'][atomic_numbers]
    w = _silu(rbf @ params['W_r1']) @ params['W_r2']
    x = x + _sc_segment_sum(w * (sh @ params['W_sh']), dst, N) / jnp.sqrt(AVG_DEG)
    x = x + _attn(_ln(x), rbf, src, dst, params['blk0'], N)
    x = _attn(_ln(x), rbf, src, dst, params['blk1'], N)
    ones = jnp.ones((N, 1), jnp.float32)
    g = _sc_segment_sum(x, labels, NG) / (jax.ops.segment_sum(ones, labels, num_segments=NG) + 1e-6)
    p_l = params['long']
    q = g @ p_l['Wq']; k = x @ p_l['Wk']; v = x @ p_l['Wv']
    Ds = D1 // H
    qe = q[gid].reshape(-1, H, Ds); ke = k[nid].reshape(-1, H, Ds)
    logit = jnp.sum(qe * ke, -1) / jnp.sqrt(float(Ds)) + jnp.where(lmask, 0.0, -1e9)[:, None]
    qn = jnp.sqrt(jnp.sum(q.reshape(NG, H, Ds) ** 2, -1))
    kmax = jnp.sqrt(jnp.max(jnp.sum(k.reshape(N, H, Ds) ** 2, -1), axis=0))
    mhat = qn * kmax[None, :] / jnp.sqrt(float(Ds))
    a = jnp.exp(logit - mhat[gid])
    u = (v[nid].reshape(-1, H, Ds) * a[..., None]).reshape(-1, D1)
    su = _sc_segment_sum(jnp.concatenate([a, u], axis=1), gid, NG)
    den = su[:, :H] + 1e-9
    num = su[:, H:]
    g = g + (num.reshape(NG, H, Ds) / den[..., None]).reshape(NG, D1) @ p_l['Wo']
    xl = g[labels]
    h = jnp.concatenate([_ln(x), _ln(xl)], axis=-1)
    node_out = _readout(h, params['Wh1'], params['Wh2'])
    energy = jax.ops.segment_sum(node_out, batch, num_segments=NM) / jnp.sqrt(AVG_NODES)
    return energy


# speedup vs baseline: 1.1748x; 1.1748x over previous
import functools

import jax
import jax.numpy as jnp
from jax import lax
from jax.experimental import pallas as pl
from jax.experimental.pallas import tpu as pltpu
from jax.experimental.pallas import tpu_sc as plsc

N = 10000; E = 320000; NG = 1000; NM = 500; E2 = 100000
D0 = 240; D1 = 256; H = 4; NB = 64; CUT = 5.0; LCUT = 9.0
AVG_DEG = 15.57930850982666; AVG_NODES = 18.03065905448718

NC = 2
NS = 16
NW = NC * NS
CH = 128


def _silu(x):
    return x * jax.nn.sigmoid(x)


def _ln(x):
    m = x.mean(-1, keepdims=True)
    v = x.var(-1, keepdims=True)
    return (x - m) / jnp.sqrt(v + 1e-5)


def _sh(u):
    x, y, z = u[:, 0], u[:, 1], u[:, 2]
    s3 = jnp.sqrt(3.0); s15 = jnp.sqrt(15.0); s5 = jnp.sqrt(5.0)
    return jnp.stack([jnp.ones_like(x), s3 * x, s3 * y, s3 * z,
                      s15 * x * y, s15 * y * z, 0.5 * s5 * (3.0 * z * z - 1.0),
                      s15 * x * z, 0.5 * s15 * (x * x - y * y)], axis=-1)


def _rbf(d):
    c = jnp.linspace(0.0, CUT, NB)
    w = CUT / NB
    return jnp.exp(-(((d[:, None] - c[None, :]) / w) ** 2))


def _round_up(a, b):
    return (a + b - 1) // b * b


def _seg_sum_body(pw, n_pad, d, vals, idx, zeros, out,
                  iv0, iv1, rv0, rv1, si0, si1, sr0, sr1, acc):
    c = lax.axis_index("c")
    s = lax.axis_index("s")
    wid = s * NC + c
    base_row = wid * (pw * CH)

    rpw = n_pad // NS
    r0 = s * rpw
    pltpu.sync_copy(zeros.at[pl.ds(r0, rpw)], acc.at[pl.ds(r0, rpw)])
    plsc.subcore_barrier()

    ivs = (iv0, iv1); rvs = (rv0, rv1); sis = (si0, si1); srs = (sr0, sr1)

    def start(g, b):
        row = base_row + g * CH
        pltpu.async_copy(idx.at[pl.ds(row, CH)], ivs[b], sis[b])
        pltpu.async_copy(vals.at[pl.ds(row, CH)], rvs[b], srs[b])

    def wait(g, b):
        row = base_row + g * CH
        pltpu.make_async_copy(idx.at[pl.ds(row, CH)], ivs[b], sis[b]).wait()
        pltpu.make_async_copy(vals.at[pl.ds(row, CH)], rvs[b], srs[b]).wait()

    def scat(b):
        pltpu.sync_copy(rvs[b], acc.at[ivs[b]], add=True)

    nit = pw // 2
    start(0, 0)

    def body(t, carry):
        g0 = t * 2
        wait(g0, 0)
        start(g0 + 1, 1)
        scat(0)
        wait(g0 + 1, 1)

        @pl.when(t < nit - 1)
        def _():
            start(g0 + 2, 0)

        scat(1)
        return carry

    lax.fori_loop(0, nit, body, 0)

    plsc.subcore_barrier()
    pltpu.sync_copy(acc.at[pl.ds(r0, rpw)], out.at[pl.ds(c * n_pad + r0, rpw)])


def _sc_segment_sum_1(values, idx, n_out):
    ep, d = values.shape
    assert d in (16, 128) and ep % (NW * CH * 2) == 0
    n_pad = _round_up(n_out, NS * 8)
    pw = ep // (NW * CH)
    zeros = jnp.zeros((n_pad, d), jnp.float32)

    mesh = plsc.VectorSubcoreMesh(core_axis_name="c", subcore_axis_name="s")
    k = pl.kernel(
        functools.partial(_seg_sum_body, pw, n_pad, d),
        mesh=mesh,
        out_type=jax.ShapeDtypeStruct((NC * n_pad, d), jnp.float32),
        scratch_types=[
            pltpu.VMEM((CH,), jnp.int32),
            pltpu.VMEM((CH,), jnp.int32),
            pltpu.VMEM((CH, d), jnp.float32),
            pltpu.VMEM((CH, d), jnp.float32),
            pltpu.SemaphoreType.DMA,
            pltpu.SemaphoreType.DMA,
            pltpu.SemaphoreType.DMA,
            pltpu.SemaphoreType.DMA,
            pltpu.VMEM_SHARED((n_pad, d), jnp.float32),
        ],
    )
    out = k(values, idx.astype(jnp.int32), zeros)
    out = out.reshape(NC, n_pad, d)
    return (out[0] + out[1])[:n_out]


def _sc_segment_sum(values, idx, n_out):
    e0, d = values.shape
    dp = 16 if d <= 16 else _round_up(d, 128)
    ep = _round_up(e0, NW * CH * 2)
    if dp != d or ep != e0:
        values = jnp.pad(values, ((0, ep - e0), (0, dp - d)))
    if ep != e0:
        idx = jnp.pad(idx, (0, ep - e0))
    idx = idx.astype(jnp.int32)
    parts = []
    for c0 in range(0, dp, 128):
        w = min(128, dp - c0)
        parts.append(_sc_segment_sum_1(values[:, c0:c0 + w], idx, n_out))
    return jnp.concatenate(parts, axis=1)[:, :d]


def _edge_attn_body(inv, qe_ref, ke_ref, vs_ref, b_ref, hs_ref, ex_ref, o_ref):
    a = jnp.exp((qe_ref[...] * ke_ref[...]) @ hs_ref[...] * inv + b_ref[...])
    u = vs_ref[...] * (a @ ex_ref[...])
    pad = o_ref.shape[1] - a.shape[1] - u.shape[1]
    parts = [a, u] + ([jnp.zeros((a.shape[0], pad), jnp.float32)] if pad else [])
    o_ref[...] = jnp.concatenate(parts, axis=1)


def _edge_attn(qe, ke, vs, bias, Ds, Dv):
    e0 = qe.shape[0]
    oc = _round_up(H + H * Dv, 128)
    hsel = jnp.repeat(jnp.eye(H, dtype=jnp.float32), Ds, axis=0)
    expand = jnp.repeat(jnp.eye(H, dtype=jnp.float32), Dv, axis=1)
    BLK = 512
    assert e0 % BLK == 0
    return pl.pallas_call(
        functools.partial(_edge_attn_body, float(Ds) ** -0.5),
        grid=(e0 // BLK,),
        in_specs=[
            pl.BlockSpec((BLK, H * Ds), lambda i: (i, 0)),
            pl.BlockSpec((BLK, H * Ds), lambda i: (i, 0)),
            pl.BlockSpec((BLK, H * Dv), lambda i: (i, 0)),
            pl.BlockSpec((BLK, H), lambda i: (i, 0)),
            pl.BlockSpec((H * Ds, H), lambda i: (0, 0)),
            pl.BlockSpec((H, H * Dv), lambda i: (0, 0)),
        ],
        out_specs=pl.BlockSpec((BLK, oc), lambda i: (i, 0)),
        out_shape=jax.ShapeDtypeStruct((e0, oc), jnp.float32),
    )(qe, ke, vs, bias, hsel, expand)


def _attn(xin, rbf, src, dst, p, n_out):
    q = xin @ p['Wq']; k = xin @ p['Wk']; v = xin @ p['Wv']
    Ds = q.shape[1] // H; Dv = v.shape[1] // H
    gate = _silu(rbf @ p['Wg1']) @ p['Wg2']
    qn = jnp.sqrt(jnp.sum(q.reshape(n_out, H, Ds) ** 2, -1))
    kmax = jnp.sqrt(jnp.max(jnp.sum(k.reshape(-1, H, Ds) ** 2, -1), axis=0))
    mhat = qn * kmax[None, :] / jnp.sqrt(float(Ds)) + jnp.max(gate, axis=0)[None, :]
    cat = _edge_attn(q[dst], k[src], v[src], gate - mhat[dst], Ds, Dv)
    su = _sc_segment_sum(cat, dst, n_out)
    den = su[:, :H] + 1e-9
    num = su[:, H:H + H * Dv]
    return (num.reshape(n_out, H, Dv) / den[..., None]).reshape(n_out, H * Dv) @ p['Wo']


def _readout_body(h_ref, w1_ref, w2_ref, o_ref):
    t = h_ref[...] @ w1_ref[...]
    t = t * jax.nn.sigmoid(t)
    o_ref[...] = t @ w2_ref[...]


def _readout(h, w1, w2):
    BLK = 400
    return pl.pallas_call(
        _readout_body,
        grid=(N // BLK,),
        in_specs=[
            pl.BlockSpec((BLK, 2 * D1), lambda i: (i, 0)),
            pl.BlockSpec((2 * D1, D1), lambda i: (0, 0)),
            pl.BlockSpec((D1, 1), lambda i: (0, 0)),
        ],
        out_specs=pl.BlockSpec((BLK, 1), lambda i: (i, 0)),
        out_shape=jax.ShapeDtypeStruct((N, 1), jnp.float32),
    )(h, w1, w2)


def kernel(pos, params, batch, labels, atomic_numbers, edge_index, interaction_graph):
    z = atomic_numbers.astype(jnp.float32)[:, None]
    gpos = jax.ops.segment_sum(pos * z, labels, num_segments=NG) / (jax.ops.segment_sum(z, labels, num_segments=NG) + 1e-6)
    nid, gid = interaction_graph[0], interaction_graph[1]
    ngd = jnp.sqrt(jnp.sum((pos[nid] - gpos[gid]) ** 2, -1) + 1e-12)
    lmask = (ngd <= LCUT)
    src, dst = edge_index[0], edge_index[1]
    evec = pos[src] - pos[dst]
    elen = jnp.sqrt(jnp.sum(evec ** 2, -1) + 1e-12)
    sh = _sh(evec / elen[:, None])
    rbf = _rbf(elen)
    x = params['W_atom'][atomic_numbers]
    w = _silu(rbf @ params['W_r1']) @ params['W_r2']
    x = x + _sc_segment_sum(w * (sh @ params['W_sh']), dst, N) / jnp.sqrt(AVG_DEG)
    x = x + _attn(_ln(x), rbf, src, dst, params['blk0'], N)
    x = _attn(_ln(x), rbf, src, dst, params['blk1'], N)
    ones = jnp.ones((N, 1), jnp.float32)
    g = _sc_segment_sum(x, labels, NG) / (jax.ops.segment_sum(ones, labels, num_segments=NG) + 1e-6)
    p_l = params['long']
    q = g @ p_l['Wq']; k = x @ p_l['Wk']; v = x @ p_l['Wv']
    Ds = D1 // H
    qe = q[gid].reshape(-1, H, Ds); ke = k[nid].reshape(-1, H, Ds)
    logit = jnp.sum(qe * ke, -1) / jnp.sqrt(float(Ds)) + jnp.where(lmask, 0.0, -1e9)[:, None]
    qn = jnp.sqrt(jnp.sum(q.reshape(NG, H, Ds) ** 2, -1))
    kmax = jnp.sqrt(jnp.max(jnp.sum(k.reshape(N, H, Ds) ** 2, -1), axis=0))
    mhat = qn * kmax[None, :] / jnp.sqrt(float(Ds))
    a = jnp.exp(logit - mhat[gid])
    u = (v[nid].reshape(-1, H, Ds) * a[..., None]).reshape(-1, D1)
    su = _sc_segment_sum(jnp.concatenate([a, u], axis=1), gid, NG)
    den = su[:, :H] + 1e-9
    num = su[:, H:]
    g = g + (num.reshape(NG, H, Ds) / den[..., None]).reshape(NG, D1) @ p_l['Wo']
    xl = g[labels]
    h = jnp.concatenate([_ln(x), _ln(xl)], axis=-1)
    node_out = _readout(h, params['Wh1'], params['Wh2'])
    energy = jax.ops.segment_sum(node_out, batch, num_segments=NM) / jnp.sqrt(AVG_NODES)
    return energy
